# Initial kernel scaffold; baseline (speedup 1.0000x reference)
import jax
import jax.numpy as jnp
from jax.experimental import pallas as pl


def kernel(features1, features2, features3, edge_index, W1a, b1a, W1b, b1b, W2a, b2a, W2b, b2b, W3a, b3a, W3b, b3b):
    raise NotImplementedError("write your pallas kernel here")


# speedup vs baseline: 14.8641x; 14.8641x over previous
import functools

import jax
import jax.numpy as jnp
from jax import lax
from jax.experimental import pallas as pl
from jax.experimental.pallas import tpu as pltpu
from jax.experimental.pallas import tpu_sc as plsc

_N = 50000
_E = 1600000
_D = 128
_H = 128
_C = 48
_ALPHA = 0.1
_K = 10
_A1, _A2, _A3 = 0.4, 0.3, 0.3

_N2 = _N // 2
_RPT = 1568
_ACC_ROWS = 16 * _RPT
_NP = 2 * _ACC_ROWS
_DUMP = _N2
_JCH = 6
_CHUNK_E = _JCH * 128
_NCHUNK = 131
_EPT = _NCHUNK * _CHUNK_E
_E_PAD = 16 * _EPT
_PCH = 14
_PR = _RPT // _PCH


def _step_body(g_hbm, a_hbm, b_hbm, src_hbm, dst_hbm, out_hbm,
               acc, srcv, dstv, rows, mv, av, bv, gsem, ssem):
    c = lax.axis_index("c")
    s = lax.axis_index("s")
    zero16 = jnp.zeros((16,), jnp.float32)

    def zrow(r, _):
        for cg in range(3):
            mv[r, pl.ds(cg * 16, 16)] = zero16
        return 0
    lax.fori_loop(0, _PR, zrow, 0)
    for cc in range(_PCH):
        pltpu.sync_copy(mv, acc.at[pl.ds(s * _RPT + cc * _PR, _PR)])
    plsc.subcore_barrier()

    def chunk(k, _):
        pltpu.sync_copy(src_hbm.at[s, k], srcv)
        pltpu.sync_copy(dst_hbm.at[c, s, k], dstv)
        gcps = [pltpu.async_copy(g_hbm.at[srcv.at[j]], rows.at[j], gsem)
                for j in range(_JCH)]
        for cp in gcps:
            cp.wait()
        scps = [pltpu.async_copy(rows.at[j], acc.at[dstv.at[j]], ssem, add=True)
                for j in range(_JCH)]
        for cp in scps:
            cp.wait()
        return 0
    lax.fori_loop(0, _NCHUNK, chunk, 0)
    plsc.subcore_barrier()

    for cc in range(_PCH):
        r0 = s * _RPT + cc * _PR
        g0 = c * _ACC_ROWS + r0
        pltpu.sync_copy(acc.at[pl.ds(r0, _PR)], mv)
        pltpu.sync_copy(a_hbm.at[pl.ds(g0, _PR)], av)
        pltpu.sync_copy(b_hbm.at[pl.ds(g0, _PR)], bv)

        def prow(r, _):
            for cg in range(3):
                sl = pl.ds(cg * 16, 16)
                mv[r, sl] = mv[r, sl] * av[r, sl] + bv[r, sl]
            return 0
        lax.fori_loop(0, _PR, prow, 0)
        pltpu.sync_copy(mv, out_hbm.at[pl.ds(g0, _PR)])


@jax.jit
def _step(g_tbl, a_tbl, b_tbl, src_arr, dst_arr):
    mesh = plsc.VectorSubcoreMesh(core_axis_name="c", subcore_axis_name="s")
    return pl.kernel(
        _step_body,
        out_type=jax.ShapeDtypeStruct((_NP, _C), jnp.float32),
        mesh=mesh,
        compiler_params=pltpu.CompilerParams(use_tc_tiling_on_sc=False),
        scratch_types=[
            pltpu.VMEM_SHARED((_ACC_ROWS, _C), jnp.float32),
            pltpu.VMEM((_JCH, 128), jnp.int32),
            pltpu.VMEM((_JCH, 128), jnp.int32),
            pltpu.VMEM((_JCH, 128, _C), jnp.float32),
            pltpu.VMEM((_PR, _C), jnp.float32),
            pltpu.VMEM((_PR, _C), jnp.float32),
            pltpu.VMEM((_PR, _C), jnp.float32),
            pltpu.SemaphoreType.DMA,
            pltpu.SemaphoreType.DMA,
        ],
    )(g_tbl, a_tbl, b_tbl, src_arr, dst_arr)


def _mlp_body(f1, f2, f3, deg,
              w1a, b1a, w1b, b1b, w2a, b2a, w2b, b2b, w3a, b3a, w3b, b3b,
              g0_o, nsq_o, ah0n_o, fina_o, finb_o):
    h = jnp.zeros_like(g0_o)
    for x_ref, wa, ba, wb, bb, aw in (
            (f1, w1a, b1a, w1b, b1b, _A1),
            (f2, w2a, b2a, w2b, b2b, _A2),
            (f3, w3a, b3a, w3b, b3b, _A3)):
        t = jnp.maximum(
            jnp.dot(x_ref[...], wa[...], preferred_element_type=jnp.float32)
            + ba[...], 0.0)
        h = h + aw * (jnp.dot(t, wb[...], preferred_element_type=jnp.float32)
                      + bb[...])
    norm = lax.rsqrt(jnp.clip(deg[...], 1.0, None))
    one_m_a = 1.0 - _ALPHA
    g0_o[...] = h * norm
    nsq_o[...] = one_m_a * norm * norm
    ah0n_o[...] = _ALPHA * h * norm
    fina_o[...] = one_m_a * norm
    finb_o[...] = _ALPHA * h


@jax.jit
def _mlp(f1p, f2p, f3p, deg48,
         w1a, b1a, w1b, b1b, w2a, b2a, w2b, b2b, w3a, b3a, w3b, b3b):
    blk = 512
    grid = (_NP // blk,)
    fspec = pl.BlockSpec((blk, _D), lambda i: (i, 0))
    dspec = pl.BlockSpec((blk, _C), lambda i: (i, 0))
    waspec = pl.BlockSpec((_D, _H), lambda i: (0, 0))
    baspec = pl.BlockSpec((1, _H), lambda i: (0, 0))
    wbspec = pl.BlockSpec((_H, _C), lambda i: (0, 0))
    bbspec = pl.BlockSpec((1, _C), lambda i: (0, 0))
    ospec = pl.BlockSpec((blk, _C), lambda i: (i, 0))
    out = jax.ShapeDtypeStruct((_NP, _C), jnp.float32)
    return pl.pallas_call(
        _mlp_body,
        grid=grid,
        in_specs=[fspec, fspec, fspec, dspec] + [waspec, baspec, wbspec, bbspec] * 3,
        out_specs=[ospec] * 5,
        out_shape=[out] * 5,
    )(f1p, f2p, f3p, deg48,
      w1a, b1a.reshape(1, _H), w1b, b1b.reshape(1, _C),
      w2a, b2a.reshape(1, _H), w2b, b2b.reshape(1, _C),
      w3a, b3a.reshape(1, _H), w3b, b3b.reshape(1, _C))


def kernel(features1, features2, features3, edge_index,
           W1a, b1a, W1b, b1b, W2a, b2a, W2b, b2b, W3a, b3a, W3b, b3b):
    src = edge_index[0].astype(jnp.int32)
    dst = edge_index[1].astype(jnp.int32)

    srcp = src + jnp.where(src >= _N2, _ACC_ROWS - _N2, 0).astype(jnp.int32)
    pad = _E_PAD - _E
    src_arr = jnp.concatenate(
        [srcp, jnp.zeros((pad,), jnp.int32)]).reshape(16, _NCHUNK, _JCH, 128)
    d0 = jnp.where(dst < _N2, dst, _DUMP)
    d1 = jnp.where(dst >= _N2, dst - _N2, _DUMP)
    dpad = jnp.full((pad,), _DUMP, jnp.int32)
    dst_arr = jnp.stack([
        jnp.concatenate([d0, dpad]),
        jnp.concatenate([d1, dpad]),
    ]).reshape(2, 16, _NCHUNK, _JCH, 128)

    ones_tbl = jnp.ones((_NP, _C), jnp.float32)
    zeros_tbl = jnp.zeros((_NP, _C), jnp.float32)
    deg48 = _step(ones_tbl, ones_tbl, zeros_tbl, src_arr, dst_arr)

    z88 = jnp.zeros((_ACC_ROWS - _N2, _D), jnp.float32)

    def remap(f):
        return jnp.concatenate([f[:_N2], z88, f[_N2:], z88], axis=0)

    g, nsq, ah0n, fina, finb = _mlp(
        remap(features1), remap(features2), remap(features3), deg48,
        W1a, b1a, W1b, b1b, W2a, b2a, W2b, b2b, W3a, b3a, W3b, b3b)

    for _ in range(_K - 1):
        g = _step(g, nsq, ah0n, src_arr, dst_arr)
    hp = _step(g, fina, finb, src_arr, dst_arr)
    return jnp.concatenate([hp[:_N2], hp[_ACC_ROWS:_ACC_ROWS + _N2]], axis=0)
